# baseline (speedup 1.0000x reference)
import functools

import jax
import jax.numpy as jnp
from jax import lax
from jax.experimental import pallas as pl
from jax.experimental.pallas import tpu as pltpu
from jax.experimental.pallas import tpu_sc as plsc

_BS = 512
_NC, _NS, _L = 2, 16, 16
_C = 8


def _add_body(x_ref, pe_ref, o_ref):
    o_ref[...] = x_ref[...] + pe_ref[...]


def _tc_add(x, pe_table, nb=None):
    B, S, D = x.shape
    if nb is None:
        nb = B
    return pl.pallas_call(
        _add_body,
        grid=(S // _BS,),
        in_specs=[
            pl.BlockSpec((nb, _BS, D), lambda i: (0, i, 0)),
            pl.BlockSpec((_BS, D), lambda i: (i, 0)),
        ],
        out_specs=pl.BlockSpec((nb, _BS, D), lambda i: (0, i, 0)),
        out_shape=jax.ShapeDtypeStruct((nb, S, D), x.dtype),
    )(x, pe_table)


def _sc_add(x, pe_table, b_lo=0, nb=None):
    full_B, S, D = x.shape
    B = full_B - b_lo if nb is None else nb
    _c = _C if B >= 3 else 16
    W = _NC * _NS
    spw = S // W
    nchunk = spw // _c
    npair = nchunk // 2
    mesh = plsc.VectorSubcoreMesh(core_axis_name="c", subcore_axis_name="s")

    @functools.partial(
        pl.kernel,
        mesh=mesh,
        out_type=jax.ShapeDtypeStruct((B, S, D), x.dtype),
        scratch_types=[
            pltpu.VMEM((2, _c, D), jnp.float32),
            pltpu.VMEM((2, B, _c, D), jnp.float32),
            pltpu.SemaphoreType.DMA((2,)),
            pltpu.SemaphoreType.DMA((2, B)),
            pltpu.SemaphoreType.DMA((2, B)),
        ],
    )
    def k(x_hbm, pe_hbm, out_hbm, pe_v, x_v, pe_sem, in_sem, out_sem):
        wid = lax.axis_index("s") * _NC + lax.axis_index("c")
        base = wid * spw

        def pe_copy_dyn(c, s):
            return pltpu.make_async_copy(
                pe_hbm.at[pl.ds(base + c * _c, _c)], pe_v.at[s], pe_sem.at[s])

        def in_copy(c, s, b):
            return pltpu.make_async_copy(
                x_hbm.at[b_lo + b, pl.ds(base + c * _c, _c)], x_v.at[s, b],
                in_sem.at[s, b])

        def out_copy(c, s, b):
            return pltpu.make_async_copy(
                x_v.at[s, b], out_hbm.at[b, pl.ds(base + c * _c, _c)],
                out_sem.at[s, b])

        pe_copy_dyn(0, 0).start()
        pe_copy_dyn(1, 1).start()
        for b in range(B):
            in_copy(0, 0, b).start()

        def do_chunk(c, s, so, first, last):
            pe_copy_dyn(c, s).wait()
            for b in range(B):
                in_copy(c, s, b).wait()

                def half_body(h, _):
                    r = h // 2
                    j0 = (h % 2) * (D // (2 * _L))
                    for j in range(D // (2 * _L)):
                        sl = pl.ds((j0 + j) * _L, _L)
                        plsc.addupdate(x_v.at[s, b, r, sl], pe_v[s, r, sl])
                    return 0

                lax.fori_loop(0, _c * 2, half_body, 0)
                out_copy(c, s, b).start()

                @pl.when(jnp.logical_not(last))
                def _():
                    @pl.when(jnp.logical_not(first))
                    def _():
                        out_copy(c - 1, so, b).wait()
                    in_copy(c + 1, so, b).start()
            @pl.when(c + 2 < nchunk)
            def _():
                pe_copy_dyn(c + 2, s).start()

        def pair_body(p, _):
            c0 = p * 2
            do_chunk(c0, 0, 1, p == 0, jnp.bool_(False))
            do_chunk(c0 + 1, 1, 0, jnp.bool_(False), p == npair - 1)
            return 0

        lax.fori_loop(0, npair, pair_body, 0)
        for b in range(B):
            out_copy(nchunk - 2, 0, b).wait()
            out_copy(nchunk - 1, 1, b).wait()

    return k(x, pe_table)


def _hybrid(x, pe_table, nb_sc):
    B, S, D = x.shape
    tc = _tc_add(x, pe_table, nb=B - nb_sc)
    sc = _sc_add(x, pe_table, b_lo=B - nb_sc, nb=nb_sc)
    return jnp.concatenate([tc, sc], axis=0)


def kernel(x, pe_table):
    return _sc_add(x, pe_table)


# speedup vs baseline: 1.9683x; 1.9683x over previous
import functools

import jax
import jax.numpy as jnp
from jax import lax
from jax.experimental import pallas as pl
from jax.experimental.pallas import tpu as pltpu
from jax.experimental.pallas import tpu_sc as plsc

_NC, _NS, _L = 2, 16, 16
_C = 8


def kernel(x, pe_table):
    B, S, D = x.shape
    W = _NC * _NS
    spw = S // W
    nchunk = spw // _C
    npair = nchunk // 2
    mesh = plsc.VectorSubcoreMesh(core_axis_name="c", subcore_axis_name="s")

    @functools.partial(
        pl.kernel,
        mesh=mesh,
        out_type=jax.ShapeDtypeStruct((B, S, D), x.dtype),
        scratch_types=[
            pltpu.VMEM((2, _C, D), jnp.float32),
            pltpu.VMEM((2, B, _C, D), jnp.float32),
            pltpu.SemaphoreType.DMA((2,)),
            pltpu.SemaphoreType.DMA((2, B)),
            pltpu.SemaphoreType.DMA((2, B)),
        ],
    )
    def k(x_hbm, pe_hbm, out_hbm, pe_v, x_v, pe_sem, in_sem, out_sem):
        wid = lax.axis_index("s") * _NC + lax.axis_index("c")
        base = wid * spw

        def pe_copy(c, s):
            return pltpu.make_async_copy(
                pe_hbm.at[pl.ds(base + c * _C, _C)], pe_v.at[s], pe_sem.at[s])

        def in_copy(c, s, b):
            return pltpu.make_async_copy(
                x_hbm.at[b, pl.ds(base + c * _C, _C)], x_v.at[s, b],
                in_sem.at[s, b])

        def out_copy(c, s, b):
            return pltpu.make_async_copy(
                x_v.at[s, b], out_hbm.at[b, pl.ds(base + c * _C, _C)],
                out_sem.at[s, b])

        pe_copy(0, 0).start()
        pe_copy(1, 1).start()
        for b in range(B):
            in_copy(0, 0, b).start()

        def do_chunk(c, s, so, first, last):
            pe_copy(c, s).wait()
            for b in range(B):
                in_copy(c, s, b).wait()

                def row_body(r, _):
                    for j in range(D // _L):
                        sl = pl.ds(j * _L, _L)
                        plsc.addupdate(x_v.at[s, b, r, sl], pe_v[s, r, sl])
                    return 0

                lax.fori_loop(0, _C, row_body, 0)
                out_copy(c, s, b).start()

                @pl.when(jnp.logical_not(last))
                def _():
                    @pl.when(jnp.logical_not(first))
                    def _():
                        out_copy(c - 1, so, b).wait()
                    in_copy(c + 1, so, b).start()
            @pl.when(c + 2 < nchunk)
            def _():
                pe_copy(c + 2, s).start()

        def pair_body(p, _):
            c0 = p * 2
            do_chunk(c0, 0, 1, p == 0, jnp.bool_(False))
            do_chunk(c0 + 1, 1, 0, jnp.bool_(False), p == npair - 1)
            return 0

        lax.fori_loop(0, npair, pair_body, 0)
        for b in range(B):
            out_copy(nchunk - 2, 0, b).wait()
            out_copy(nchunk - 1, 1, b).wait()

    return k(x, pe_table)
